# baseline (speedup 1.0000x reference)
import functools

import jax
import jax.numpy as jnp
from jax import lax
from jax.experimental import pallas as pl
from jax.experimental.pallas import tpu as pltpu
from jax.experimental.pallas import tpu_sc as plsc

NC = 2
NS = 16
NW = NC * NS

N_NODES = 10000
D = 128
CHUNK = 128
ROWS_PER_SUB = 640
NPAD = NS * ROWS_PER_SUB
ZROWS = 32
ZCOPIES = ROWS_PER_SUB // ZROWS
IW = 8

def _mesh():
    return plsc.VectorSubcoreMesh(core_axis_name="c", subcore_axis_name="s")


def _deg_body(dst_hbm, out_hbm, dstv, onesv, zbuf, acc, sem):
    cid = lax.axis_index("c")
    sid = lax.axis_index("s")
    wid = sid * NC + cid
    nch = dst_hbm.shape[1] - IW
    pltpu.sync_copy(dst_hbm.at[wid], dstv)

    def fill(r, _):
        onesv[r, :] = jnp.full((16,), 1.0, jnp.float32)
        return 0
    lax.fori_loop(0, CHUNK, fill, 0)

    def zfill(r, _):
        zbuf[r, :] = jnp.zeros((16,), jnp.float32)
        return 0
    lax.fori_loop(0, ZROWS, zfill, 0)

    def zcopy(k, _):
        pltpu.sync_copy(zbuf, acc.at[pl.ds(sid * ROWS_PER_SUB + k * ZROWS, ZROWS)])
        return 0
    lax.fori_loop(0, ZCOPIES, zcopy, 0)
    plsc.subcore_barrier()

    def step(j, _):
        pltpu.sync_copy(onesv, acc.at[dstv.at[j]], add=True)
        return 0
    lax.fori_loop(0, nch, step, 0)
    plsc.subcore_barrier()

    pltpu.sync_copy(acc.at[pl.ds(sid * ROWS_PER_SUB, ROWS_PER_SUB)],
                    out_hbm.at[cid, pl.ds(sid * ROWS_PER_SUB, ROWS_PER_SUB)])


def _segsum_body(h_hbm, src_hbm, dst_hbm, out_hbm, srcw, dstv, buf0, buf1,
                 acc, semg0, semg1):
    cid = lax.axis_index("c")
    sid = lax.axis_index("s")
    wid = sid * NC + cid
    nwin = (src_hbm.shape[1] - IW) // IW

    pltpu.sync_copy(dst_hbm.at[wid], dstv)

    def zfill(r, _):
        for c in range(D // 16):
            buf0[r, pl.ds(c * 16, 16)] = jnp.zeros((16,), jnp.float32)
        return 0
    lax.fori_loop(0, CHUNK, zfill, 0)

    def zcopy(k, _):
        pltpu.sync_copy(buf0, acc.at[pl.ds(sid * ROWS_PER_SUB + k * CHUNK, CHUNK)])
        return 0
    lax.fori_loop(0, ROWS_PER_SUB // CHUNK, zcopy, 0)
    plsc.subcore_barrier()

    bufs = (buf0, buf1)
    semg = (semg0, semg1)

    def win(w, _):
        pltpu.sync_copy(src_hbm.at[wid, pl.ds(w * IW, IW)], srcw)
        cp = pltpu.async_copy(h_hbm.at[srcw.at[0]], buf0, semg0)
        for c in range(IW):
            b = c % 2
            cp.wait()
            if c + 1 < IW:
                cp = pltpu.async_copy(h_hbm.at[srcw.at[c + 1]], bufs[1 - b],
                                      semg[1 - b])
            pltpu.sync_copy(bufs[b], acc.at[dstv.at[w * IW + c]], add=True)
        return 0

    lax.fori_loop(0, nwin, win, 0)
    plsc.subcore_barrier()

    pltpu.sync_copy(acc.at[pl.ds(sid * ROWS_PER_SUB, ROWS_PER_SUB)],
                    out_hbm.at[cid, pl.ds(sid * ROWS_PER_SUB, ROWS_PER_SUB)])


@functools.lru_cache(maxsize=None)
def _make_deg_kernel(n_chunks):
    return pl.kernel(
        _deg_body,
        mesh=_mesh(),
        out_type=jax.ShapeDtypeStruct((NC, NPAD, 16), jnp.float32),
        scratch_types=[
            pltpu.VMEM((n_chunks, CHUNK), jnp.int32),
            pltpu.VMEM((CHUNK, 16), jnp.float32),
            pltpu.VMEM((ZROWS, 16), jnp.float32),
            pltpu.VMEM_SHARED((NPAD, 16), jnp.float32),
            pltpu.SemaphoreType.DMA,
        ],
    )


@functools.lru_cache(maxsize=None)
def _make_segsum_kernel(n_chunks):
    return pl.kernel(
        _segsum_body,
        mesh=_mesh(),
        out_type=jax.ShapeDtypeStruct((NC, NPAD, D), jnp.float32),
        scratch_types=[
            pltpu.VMEM((IW, CHUNK), jnp.int32),
            pltpu.VMEM((n_chunks, CHUNK), jnp.int32),
            pltpu.VMEM((CHUNK, D), jnp.float32),
            pltpu.VMEM((CHUNK, D), jnp.float32),
            pltpu.VMEM_SHARED((NPAD, D), jnp.float32),
            pltpu.SemaphoreType.DMA,
            pltpu.SemaphoreType.DMA,
        ],
    )


ROWS_BLK = 1000
GRID = N_NODES // ROWS_BLK


def _dinv_from(degp_ref):
    deg = degp_ref[0, :, 0:1] + degp_ref[1, :, 0:1] + 1.0
    return lax.rsqrt(deg)


def _tc1_body(x_ref, w_ref, degp_ref, out_ref):
    dinv = _dinv_from(degp_ref)
    out_ref[...] = jnp.dot(x_ref[...], w_ref[...],
                           preferred_element_type=jnp.float32) * dinv


def _tc2_body(p_ref, hp_ref, degp_ref, b_ref, w_ref, out_ref):
    dinv = _dinv_from(degp_ref)
    z = dinv * (p_ref[0] + p_ref[1] + hp_ref[...]) + b_ref[...]
    z = jnp.maximum(z, 0.0)
    out_ref[...] = jnp.dot(z, w_ref[...],
                           preferred_element_type=jnp.float32) * dinv


def _tc3_body(q_ref, hp_ref, degp_ref, b_ref, out_ref):
    dinv = _dinv_from(degp_ref)
    z = dinv * (q_ref[0] + q_ref[1] + hp_ref[...]) + b_ref[...]
    m = jnp.max(z, axis=-1, keepdims=True)
    e = jnp.exp(z - m)
    s = jnp.sum(e, axis=-1, keepdims=True)
    out_ref[...] = (z - m) - jnp.log(s)


_row_spec = pl.BlockSpec((ROWS_BLK, D), lambda i: (i, 0))
_deg_spec = pl.BlockSpec((NC, ROWS_BLK, 16), lambda i: (0, i, 0))
_par_spec = pl.BlockSpec((NC, ROWS_BLK, D), lambda i: (0, i, 0))
_w_spec = pl.BlockSpec((D, D), lambda i: (0, 0))
_b_spec = pl.BlockSpec((1, D), lambda i: (0, 0))

_tc1 = pl.pallas_call(
    _tc1_body,
    grid=(GRID,),
    in_specs=[_row_spec, _w_spec, _deg_spec],
    out_specs=_row_spec,
    out_shape=jax.ShapeDtypeStruct((N_NODES, D), jnp.float32),
)

_tc2 = pl.pallas_call(
    _tc2_body,
    grid=(GRID,),
    in_specs=[_par_spec, _row_spec, _deg_spec, _b_spec, _w_spec],
    out_specs=_row_spec,
    out_shape=jax.ShapeDtypeStruct((N_NODES, D), jnp.float32),
)

_tc3 = pl.pallas_call(
    _tc3_body,
    grid=(GRID,),
    in_specs=[_par_spec, _row_spec, _deg_spec, _b_spec],
    out_specs=_row_spec,
    out_shape=jax.ShapeDtypeStruct((N_NODES, D), jnp.float32),
)


def kernel(X, edges, W1, b1, W2, b2):
    n = X.shape[0]
    e = edges.shape[1]
    src = edges[0].astype(jnp.int32)
    dst = edges[1].astype(jnp.int32)

    n_chunks = -(-e // (NW * CHUNK))
    n_chunks = -(-n_chunks // 8) * 8
    ep = NW * n_chunks * CHUNK
    padn = ep - e
    pad_i = jnp.arange(padn, dtype=jnp.int32)
    pad_src = (pad_i * 37) % n
    pad_dst = n + pad_i % (NPAD - n)
    src_t = jnp.concatenate([src, pad_src]).reshape(NW, n_chunks, CHUNK)
    dst_t = jnp.concatenate([dst, pad_dst]).reshape(NW, n_chunks, CHUNK)
    src_t = jnp.concatenate(
        [src_t, jnp.zeros((NW, IW, CHUNK), jnp.int32)], axis=1)
    dst_t = jnp.concatenate(
        [dst_t, jnp.full((NW, IW, CHUNK), n, jnp.int32)], axis=1)
    n_chunks += IW

    deg_kernel = _make_deg_kernel(n_chunks)
    segsum_kernel = _make_segsum_kernel(n_chunks)
    degp = deg_kernel(dst_t)
    hp1 = _tc1(X, W1, degp)
    p = segsum_kernel(hp1, src_t, dst_t)
    hp2 = _tc2(p, hp1, degp, b1.reshape(1, D), W2)
    q = segsum_kernel(hp2, src_t, dst_t)
    return _tc3(q, hp2, degp, b2.reshape(1, D))


# speedup vs baseline: 1.1941x; 1.1941x over previous
import functools

import jax
import jax.numpy as jnp
from jax import lax
from jax.experimental import pallas as pl
from jax.experimental.pallas import tpu as pltpu
from jax.experimental.pallas import tpu_sc as plsc

NC = 2
NS = 16
NW = NC * NS

N_NODES = 10000
D = 128
CHUNK = 128
ROWS_PER_SUB = 640
NPAD = NS * ROWS_PER_SUB
ZROWS = 32
ZCOPIES = ROWS_PER_SUB // ZROWS
IW = 8

def _mesh():
    return plsc.VectorSubcoreMesh(core_axis_name="c", subcore_axis_name="s")


def _deg_body(dst_hbm, out_hbm, dstv, onesv, zbuf, acc, sem):
    cid = lax.axis_index("c")
    sid = lax.axis_index("s")
    wid = sid * NC + cid
    nch = dst_hbm.shape[1]
    pltpu.sync_copy(dst_hbm.at[wid], dstv)

    def fill(r, _):
        onesv[r, :] = jnp.full((16,), 1.0, jnp.float32)
        return 0
    lax.fori_loop(0, CHUNK, fill, 0)

    def zfill(r, _):
        zbuf[r, :] = jnp.zeros((16,), jnp.float32)
        return 0
    lax.fori_loop(0, ZROWS, zfill, 0)

    def zcopy(k, _):
        pltpu.sync_copy(zbuf, acc.at[pl.ds(sid * ROWS_PER_SUB + k * ZROWS, ZROWS)])
        return 0
    lax.fori_loop(0, ZCOPIES, zcopy, 0)
    plsc.subcore_barrier()

    def step(j, _):
        pltpu.sync_copy(onesv, acc.at[dstv.at[j]], add=True)
        return 0
    lax.fori_loop(0, nch, step, 0)
    plsc.subcore_barrier()

    pltpu.sync_copy(acc.at[pl.ds(sid * ROWS_PER_SUB, ROWS_PER_SUB)],
                    out_hbm.at[cid, pl.ds(sid * ROWS_PER_SUB, ROWS_PER_SUB)])


_PMASK = (1 << 14) - 1


def _segsum_body(h_hbm, pidx_hbm, out_hbm, pidxv, srcst, dstst, buf0, buf1,
                 acc, semg0, semg1, sems0, sems1):
    cid = lax.axis_index("c")
    sid = lax.axis_index("s")
    wid = sid * NC + cid
    nch = pidx_hbm.shape[1]

    pltpu.sync_copy(pidx_hbm.at[wid], pidxv)

    def zfill(r, _):
        for c in range(D // 16):
            buf0[r, pl.ds(c * 16, 16)] = jnp.zeros((16,), jnp.float32)
        return 0
    lax.fori_loop(0, CHUNK, zfill, 0)

    def zcopy(k, _):
        pltpu.sync_copy(buf0, acc.at[pl.ds(sid * ROWS_PER_SUB + k * CHUNK, CHUNK)])
        return 0
    lax.fori_loop(0, ROWS_PER_SUB // CHUNK, zcopy, 0)
    plsc.subcore_barrier()

    bufs = (buf0, buf1)
    semg = (semg0, semg1)
    sems = (sems0, sems1)

    def unpack(c):
        r = c % 8
        for k in range(CHUNK // 16):
            v = pidxv[c, pl.ds(k * 16, 16)]
            srcst[r, pl.ds(k * 16, 16)] = v & _PMASK
            dstst[r, pl.ds(k * 16, 16)] = lax.shift_right_logical(v, 14)

    unpack(0)
    unpack(1)
    g = [None] * nch
    s = [None] * nch
    g[0] = pltpu.async_copy(h_hbm.at[srcst.at[0]], buf0, semg0)
    g[1] = pltpu.async_copy(h_hbm.at[srcst.at[1]], buf1, semg1)
    for c in range(nch):
        b = c % 2
        g[c].wait()
        s[c] = pltpu.async_copy(bufs[b], acc.at[dstst.at[c % 8]], sems[b],
                                add=True)
        if c + 2 < nch:
            unpack(c + 2)
            s[c].wait()
            g[c + 2] = pltpu.async_copy(h_hbm.at[srcst.at[(c + 2) % 8]],
                                        bufs[b], semg[b])
    s[nch - 2].wait()
    s[nch - 1].wait()
    plsc.subcore_barrier()

    pltpu.sync_copy(acc.at[pl.ds(sid * ROWS_PER_SUB, ROWS_PER_SUB)],
                    out_hbm.at[cid, pl.ds(sid * ROWS_PER_SUB, ROWS_PER_SUB)])


@functools.lru_cache(maxsize=None)
def _make_deg_kernel(n_chunks):
    return pl.kernel(
        _deg_body,
        mesh=_mesh(),
        out_type=jax.ShapeDtypeStruct((NC, NPAD, 16), jnp.float32),
        scratch_types=[
            pltpu.VMEM((n_chunks, CHUNK), jnp.int32),
            pltpu.VMEM((CHUNK, 16), jnp.float32),
            pltpu.VMEM((ZROWS, 16), jnp.float32),
            pltpu.VMEM_SHARED((NPAD, 16), jnp.float32),
            pltpu.SemaphoreType.DMA,
        ],
    )


@functools.lru_cache(maxsize=None)
def _make_segsum_kernel(n_chunks):
    return pl.kernel(
        _segsum_body,
        mesh=_mesh(),
        out_type=jax.ShapeDtypeStruct((NC, NPAD, D), jnp.float32),
        scratch_types=[
            pltpu.VMEM((n_chunks, CHUNK), jnp.int32),
            pltpu.VMEM((8, CHUNK), jnp.int32),
            pltpu.VMEM((8, CHUNK), jnp.int32),
            pltpu.VMEM((CHUNK, D), jnp.float32),
            pltpu.VMEM((CHUNK, D), jnp.float32),
            pltpu.VMEM_SHARED((NPAD, D), jnp.float32),
            pltpu.SemaphoreType.DMA,
            pltpu.SemaphoreType.DMA,
            pltpu.SemaphoreType.DMA,
            pltpu.SemaphoreType.DMA,
        ],
    )


ROWS_BLK = 1000
GRID = N_NODES // ROWS_BLK


def _dinv_from(degp_ref):
    deg = degp_ref[0, :, 0:1] + degp_ref[1, :, 0:1] + 1.0
    return lax.rsqrt(deg)


def _tc1_body(x_ref, w_ref, degp_ref, out_ref):
    dinv = _dinv_from(degp_ref)
    out_ref[...] = jnp.dot(x_ref[...], w_ref[...],
                           preferred_element_type=jnp.float32) * dinv


def _tc2_body(p_ref, hp_ref, degp_ref, b_ref, w_ref, out_ref):
    dinv = _dinv_from(degp_ref)
    z = dinv * (p_ref[0] + p_ref[1] + hp_ref[...]) + b_ref[...]
    z = jnp.maximum(z, 0.0)
    out_ref[...] = jnp.dot(z, w_ref[...],
                           preferred_element_type=jnp.float32) * dinv


def _tc3_body(q_ref, hp_ref, degp_ref, b_ref, out_ref):
    dinv = _dinv_from(degp_ref)
    z = dinv * (q_ref[0] + q_ref[1] + hp_ref[...]) + b_ref[...]
    m = jnp.max(z, axis=-1, keepdims=True)
    e = jnp.exp(z - m)
    s = jnp.sum(e, axis=-1, keepdims=True)
    out_ref[...] = (z - m) - jnp.log(s)


_row_spec = pl.BlockSpec((ROWS_BLK, D), lambda i: (i, 0))
_deg_spec = pl.BlockSpec((NC, ROWS_BLK, 16), lambda i: (0, i, 0))
_par_spec = pl.BlockSpec((NC, ROWS_BLK, D), lambda i: (0, i, 0))
_w_spec = pl.BlockSpec((D, D), lambda i: (0, 0))
_b_spec = pl.BlockSpec((1, D), lambda i: (0, 0))

_tc1 = pl.pallas_call(
    _tc1_body,
    grid=(GRID,),
    in_specs=[_row_spec, _w_spec, _deg_spec],
    out_specs=_row_spec,
    out_shape=jax.ShapeDtypeStruct((N_NODES, D), jnp.float32),
)

_tc2 = pl.pallas_call(
    _tc2_body,
    grid=(GRID,),
    in_specs=[_par_spec, _row_spec, _deg_spec, _b_spec, _w_spec],
    out_specs=_row_spec,
    out_shape=jax.ShapeDtypeStruct((N_NODES, D), jnp.float32),
)

_tc3 = pl.pallas_call(
    _tc3_body,
    grid=(GRID,),
    in_specs=[_par_spec, _row_spec, _deg_spec, _b_spec],
    out_specs=_row_spec,
    out_shape=jax.ShapeDtypeStruct((N_NODES, D), jnp.float32),
)


def kernel(X, edges, W1, b1, W2, b2):
    n = X.shape[0]
    e = edges.shape[1]
    src = edges[0].astype(jnp.int32)
    dst = edges[1].astype(jnp.int32)

    n_chunks = -(-e // (NW * CHUNK))
    n_chunks = -(-n_chunks // 8) * 8
    ep = NW * n_chunks * CHUNK
    padn = ep - e
    pad_i = jnp.arange(padn, dtype=jnp.int32)
    pad_src = (pad_i * 37) % n
    pad_dst = n + pad_i % (NPAD - n)
    src_t = jnp.concatenate([src, pad_src]).reshape(NW, n_chunks, CHUNK)
    dst_t = jnp.concatenate([dst, pad_dst]).reshape(NW, n_chunks, CHUNK)
    pidx_t = src_t | (dst_t << 14)

    deg_kernel = _make_deg_kernel(n_chunks)
    segsum_kernel = _make_segsum_kernel(n_chunks)
    degp = deg_kernel(dst_t)
    hp1 = _tc1(X, W1, degp)
    p = segsum_kernel(hp1, pidx_t)
    hp2 = _tc2(p, hp1, degp, b1.reshape(1, D), W2)
    q = segsum_kernel(hp2, pidx_t)
    return _tc3(q, hp2, degp, b2.reshape(1, D))
